# baseline (speedup 1.0000x reference)
import functools

import jax
import jax.numpy as jnp
from jax import lax
from jax.experimental import pallas as pl
from jax.experimental.pallas import tpu as pltpu
from jax.experimental.pallas import tpu_sc as plsc

N = 10000
E = 320000
G = 8
F = 128
M_DIM = 16
EDGE_HID = 514
NODE_HID = 256

NC = 2
NS = 16
NW = NC * NS
ET = E // NW
CH = 128
NCH = 80
ETP = NCH * CH
EP = NW * ETP
NA = 10240
STRIPE = NA // NS

_f32 = jnp.float32


def _silu(v):
    return v / (1.0 + jnp.exp(-v))


@functools.cache
def _make_gather(feat_dim, dtype):
    mesh = plsc.VectorSubcoreMesh(core_axis_name="c", subcore_axis_name="s", num_cores=NC, num_subcores=NS)

    def body(table, src_i, dst_i, out_a, out_b, idx_s, idx_d,
             a0, a1, b0, b1, sa0, sa1, sb0, sb1):
        c = lax.axis_index("c")
        s = lax.axis_index("s")
        w = s * NC + c
        bufs_a = (a0, a1)
        bufs_b = (b0, b1)
        sems_a = (sa0, sa1)
        sems_b = (sb0, sb1)
        pltpu.sync_copy(src_i.at[w], idx_s)
        pltpu.sync_copy(dst_i.at[w], idx_d)

        def start(j, buf):
            pltpu.async_copy(table.at[idx_s.at[j]], bufs_a[buf], sems_a[buf])
            pltpu.async_copy(table.at[idx_d.at[j]], bufs_b[buf], sems_b[buf])

        def finish(j, buf):
            pltpu.make_async_copy(table.at[idx_s.at[j]], bufs_a[buf],
                                  sems_a[buf]).wait()
            pltpu.sync_copy(bufs_a[buf], out_a.at[w, j])
            pltpu.make_async_copy(table.at[idx_d.at[j]], bufs_b[buf],
                                  sems_b[buf]).wait()
            pltpu.sync_copy(bufs_b[buf], out_b.at[w, j])

        start(0, 0)

        def pair(p, carry):
            j0 = 2 * p
            start(j0 + 1, 1)
            finish(j0, 0)

            @pl.when(p < NCH // 2 - 1)
            def _():
                start(j0 + 2, 0)

            finish(j0 + 1, 1)
            return carry

        lax.fori_loop(0, NCH // 2, pair, 0)

    out = jax.ShapeDtypeStruct((NW, NCH, CH, feat_dim), dtype)
    return pl.kernel(
        body,
        out_type=[out, out],
        mesh=mesh,
        scratch_types=[
            pltpu.VMEM((NCH, CH), jnp.int32),
            pltpu.VMEM((NCH, CH), jnp.int32),
            pltpu.VMEM((CH, feat_dim), dtype),
            pltpu.VMEM((CH, feat_dim), dtype),
            pltpu.VMEM((CH, feat_dim), dtype),
            pltpu.VMEM((CH, feat_dim), dtype),
            pltpu.SemaphoreType.DMA,
            pltpu.SemaphoreType.DMA,
            pltpu.SemaphoreType.DMA,
            pltpu.SemaphoreType.DMA,
        ],
        compiler_params=pltpu.CompilerParams(use_tc_tiling_on_sc=False),
    )


@functools.cache
def _make_scatter():
    mesh = plsc.VectorSubcoreMesh(core_axis_name="c", subcore_axis_name="s", num_cores=NC, num_subcores=NS)

    def body(m3, dst_i, zeros, out, acc, idx_all, m0, m1, sm0, sm1):
        c = lax.axis_index("c")
        s = lax.axis_index("s")
        w = s * NC + c
        mbufs = (m0, m1)
        msems = (sm0, sm1)
        pltpu.sync_copy(zeros.at[pl.ds(s * STRIPE, STRIPE)],
                        acc.at[pl.ds(s * STRIPE, STRIPE)])
        pltpu.sync_copy(dst_i.at[w], idx_all)
        plsc.subcore_barrier()

        def start(j, buf):
            pltpu.async_copy(m3.at[w, j], mbufs[buf], msems[buf])

        def finish(j, buf):
            pltpu.make_async_copy(m3.at[w, j], mbufs[buf], msems[buf]).wait()
            pltpu.sync_copy(mbufs[buf], acc.at[idx_all.at[j]], add=True)

        start(0, 0)

        def pair(p, carry):
            j0 = 2 * p
            start(j0 + 1, 1)
            finish(j0, 0)

            @pl.when(p < NCH // 2 - 1)
            def _():
                start(j0 + 2, 0)

            finish(j0 + 1, 1)
            return carry

        lax.fori_loop(0, NCH // 2, pair, 0)
        plsc.subcore_barrier()
        pltpu.sync_copy(acc.at[pl.ds(s * STRIPE, STRIPE)],
                        out.at[c, pl.ds(s * STRIPE, STRIPE)])

    return pl.kernel(
        body,
        out_type=jax.ShapeDtypeStruct((NC, NA, M_DIM), _f32),
        mesh=mesh,
        scratch_types=[
            pltpu.VMEM_SHARED((NA, M_DIM), _f32),
            pltpu.VMEM((NCH, CH), jnp.int32),
            pltpu.VMEM((CH, M_DIM), _f32),
            pltpu.VMEM((CH, M_DIM), _f32),
            pltpu.SemaphoreType.DMA,
            pltpu.SemaphoreType.DMA,
        ],
        compiler_params=pltpu.CompilerParams(use_tc_tiling_on_sc=False),
    )


_BN = 400


def _lin0_body(x_ref, w_ref, b_ref, o_ref, ob_ref):
    o = jnp.maximum(
        jnp.dot(x_ref[...], w_ref[...], preferred_element_type=_f32)
        + b_ref[...], 0.0)
    o_ref[...] = o
    ob_ref[...] = o.astype(jnp.bfloat16)


def _lin0(x, w, b):
    return pl.pallas_call(
        _lin0_body,
        grid=(N // _BN,),
        in_specs=[
            pl.BlockSpec((_BN, F), lambda i: (i, 0)),
            pl.BlockSpec((F, F), lambda i: (0, 0)),
            pl.BlockSpec((1, F), lambda i: (0, 0)),
        ],
        out_specs=[pl.BlockSpec((_BN, F), lambda i: (i, 0))] * 2,
        out_shape=[jax.ShapeDtypeStruct((N, F), _f32),
                   jax.ShapeDtypeStruct((N, F), jnp.bfloat16)],
    )(x, w, b.reshape(1, F))


_CB = 4
_BE = _CB * CH


def _edge_body(use_relu, fd_ref, fs_ref, cs_ref, cd_ref, w1d_ref, w1s_ref,
               wl_ref, b1_ref, w2_ref, b2_ref, o_ref):
    fd = fd_ref[...].reshape(_BE, F)
    fs = fs_ref[...].reshape(_BE, F)
    cs = cs_ref[...].reshape(_BE, 16)
    cd = cd_ref[...].reshape(_BE, 16)
    if use_relu:
        cs = jnp.maximum(cs, 0.0)
        cd = jnp.maximum(cd, 0.0)
    d = cs - cd
    rel = jnp.sum(d * d, axis=1, keepdims=True)
    relb = rel.astype(jnp.bfloat16).astype(_f32)
    wlb = wl_ref[...].astype(jnp.bfloat16).astype(_f32)
    w1d = w1d_ref[...].astype(jnp.bfloat16)
    w1s = w1s_ref[...].astype(jnp.bfloat16)
    m1 = (jnp.dot(fd, w1d, preferred_element_type=_f32)
          + jnp.dot(fs, w1s, preferred_element_type=_f32)
          + relb * wlb + b1_ref[...])
    m1 = _silu(m1)
    m2 = jnp.dot(m1.astype(jnp.bfloat16),
                 w2_ref[...].astype(jnp.bfloat16),
                 preferred_element_type=_f32) + b2_ref[...]
    o_ref[...] = _silu(m2).reshape(1, _CB, CH, M_DIM)


def _edge(use_relu, fd4, fs4, cs4, cd4, ew1, eb1, ew2, eb2):
    body = functools.partial(_edge_body, use_relu)
    return pl.pallas_call(
        body,
        grid=(NW, NCH // _CB),
        in_specs=[
            pl.BlockSpec((1, _CB, CH, F), lambda w, j: (w, j, 0, 0)),
            pl.BlockSpec((1, _CB, CH, F), lambda w, j: (w, j, 0, 0)),
            pl.BlockSpec((1, _CB, CH, 16), lambda w, j: (w, j, 0, 0)),
            pl.BlockSpec((1, _CB, CH, 16), lambda w, j: (w, j, 0, 0)),
            pl.BlockSpec((F, EDGE_HID), lambda w, j: (0, 0)),
            pl.BlockSpec((F, EDGE_HID), lambda w, j: (0, 0)),
            pl.BlockSpec((1, EDGE_HID), lambda w, j: (0, 0)),
            pl.BlockSpec((1, EDGE_HID), lambda w, j: (0, 0)),
            pl.BlockSpec((EDGE_HID, M_DIM), lambda w, j: (0, 0)),
            pl.BlockSpec((1, M_DIM), lambda w, j: (0, 0)),
        ],
        out_specs=pl.BlockSpec((1, _CB, CH, M_DIM), lambda w, j: (w, j, 0, 0)),
        out_shape=jax.ShapeDtypeStruct((NW, NCH, CH, M_DIM), _f32),
    )(fd4, fs4, cs4, cd4, ew1[:F], ew1[F:2 * F], ew1[2 * F:2 * F + 1],
      eb1.reshape(1, EDGE_HID), ew2, eb2.reshape(1, M_DIM))


def _node_body(h_ref, mi_ref, w1f_ref, w1m_ref, b1_ref, w2_ref, b2_ref,
               o_ref, ob_ref):
    h = h_ref[...]
    m = mi_ref[0] + mi_ref[1]
    u = _silu(jnp.dot(h, w1f_ref[...], preferred_element_type=_f32)
              + jnp.dot(m, w1m_ref[...], preferred_element_type=_f32)
              + b1_ref[...])
    u = jnp.dot(u, w2_ref[...], preferred_element_type=_f32) + b2_ref[...]
    o = jnp.maximum(h + u, 0.0)
    o_ref[...] = o
    ob_ref[...] = o.astype(jnp.bfloat16)


def _node(h, mi, nw1, nb1, nw2, nb2):
    return pl.pallas_call(
        _node_body,
        grid=(N // _BN,),
        in_specs=[
            pl.BlockSpec((_BN, F), lambda i: (i, 0)),
            pl.BlockSpec((NC, _BN, M_DIM), lambda i: (0, i, 0)),
            pl.BlockSpec((F, NODE_HID), lambda i: (0, 0)),
            pl.BlockSpec((M_DIM, NODE_HID), lambda i: (0, 0)),
            pl.BlockSpec((1, NODE_HID), lambda i: (0, 0)),
            pl.BlockSpec((NODE_HID, F), lambda i: (0, 0)),
            pl.BlockSpec((1, F), lambda i: (0, 0)),
        ],
        out_specs=[pl.BlockSpec((_BN, F), lambda i: (i, 0))] * 2,
        out_shape=[jax.ShapeDtypeStruct((N, F), _f32),
                   jax.ShapeDtypeStruct((N, F), jnp.bfloat16)],
    )(h, mi, nw1[:F], nw1[F:], nb1.reshape(1, NODE_HID), nw2,
      nb2.reshape(1, F))


def _pool_body(h_ref, b_ref, o_ref):
    i = pl.program_id(0)

    @pl.when(i == 0)
    def _():
        o_ref[...] = jnp.zeros_like(o_ref)

    seg = b_ref[0]
    mask = (lax.broadcasted_iota(jnp.int32, (G, _BN), 0) == seg).astype(_f32)
    o_ref[...] += jnp.dot(mask, h_ref[...], preferred_element_type=_f32)


def _pool(h, batch3):
    return pl.pallas_call(
        _pool_body,
        grid=(N // _BN,),
        in_specs=[
            pl.BlockSpec((_BN, F), lambda i: (i, 0)),
            pl.BlockSpec((1, 1, _BN), lambda i: (i, 0, 0)),
        ],
        out_specs=pl.BlockSpec((G, F), lambda i: (0, 0)),
        out_shape=jax.ShapeDtypeStruct((G, F), _f32),
    )(h, batch3)


def _head_body(p0_ref, p1_ref, w1a_ref, w1b_ref, b1_ref, w2_ref, b2_ref,
               o_ref):
    z = (jnp.dot(p0_ref[...], w1a_ref[...], preferred_element_type=_f32)
         + jnp.dot(p1_ref[...], w1b_ref[...], preferred_element_type=_f32)
         + b1_ref[...])
    z = jnp.maximum(z, 0.0)
    o_ref[...] = jnp.dot(z, w2_ref[...], preferred_element_type=_f32) \
        + b2_ref[...]


def _head(p0, p1, w1, b1, w2, b2):
    return pl.pallas_call(
        _head_body,
        out_shape=jax.ShapeDtypeStruct((G, 1), _f32),
    )(p0, p1, w1[:F], w1[F:], b1.reshape(1, F), w2, b2.reshape(1, 1))


def _pad_idx(v, pad_val):
    v2 = v.reshape(NW, ET)
    v2 = jnp.pad(v2, ((0, 0), (0, ETP - ET)), constant_values=pad_val)
    return v2.reshape(NW, NCH, CH)


def kernel(x, coord, edge_index, batch, lin0_W, lin0_b, edge_W1, edge_b1,
           edge_W2, edge_b2, node_W1, node_b1, node_W2, node_b2, lin1_W,
           lin1_b, lin2_W, lin2_b):
    h0, h0b = _lin0(x, lin0_W, lin0_b)
    coordp = jnp.pad(coord, ((0, 0), (0, 13)))
    zeros_acc = jnp.zeros((NA, M_DIM), _f32)
    batch3 = batch.astype(jnp.int32).reshape(N // _BN, 1, _BN)

    pooled = []
    for a in range(2):
        src = edge_index[a, 0].astype(jnp.int32)
        dst = edge_index[a, 1].astype(jnp.int32)
        src_g = _pad_idx(src, 0)
        dst_g = _pad_idx(dst, 0)
        dst_s = _pad_idx(dst, N)

        cs4, cd4 = _make_gather(16, _f32)(coordp, src_g, dst_g)

        h, hb = h0, h0b
        for l in range(3):
            i = a * 3 + l
            fs4, fd4 = _make_gather(F, jnp.bfloat16)(hb, src_g, dst_g)
            m4 = _edge(l > 0, fd4, fs4, cs4, cd4,
                       edge_W1[i], edge_b1[i], edge_W2[i], edge_b2[i])
            parts = _make_scatter()(m4, dst_s, zeros_acc)
            h, hb = _node(h, parts, node_W1[i], node_b1[i],
                          node_W2[i], node_b2[i])
        pooled.append(_pool(h, batch3))

    return _head(pooled[0], pooled[1], lin1_W, lin1_b, lin2_W, lin2_b)


# speedup vs baseline: 1.2064x; 1.2064x over previous
import functools

import jax
import jax.numpy as jnp
from jax import lax
from jax.experimental import pallas as pl
from jax.experimental.pallas import tpu as pltpu
from jax.experimental.pallas import tpu_sc as plsc

N = 10000
E = 320000
G = 8
F = 128
M_DIM = 16
EDGE_HID = 514
NODE_HID = 256

NC = 2
NS = 16
NW = NC * NS
ET = E // NW
CH = 128
NCH = 80
ETP = NCH * CH
EP = NW * ETP
NA = 10240
STRIPE = NA // NS

_f32 = jnp.float32


def _silu(v):
    return v / (1.0 + jnp.exp(-v))


@functools.cache
def _make_gather(feat_dim, dtype, tiled=False):
    mesh = plsc.VectorSubcoreMesh(core_axis_name="c", subcore_axis_name="s", num_cores=NC, num_subcores=NS)

    def body(table, src_i, dst_i, out_a, out_b, idx_s, idx_d,
             a0, a1, b0, b1, sa0, sa1, sb0, sb1):
        c = lax.axis_index("c")
        s = lax.axis_index("s")
        w = s * NC + c
        bufs_a = (a0, a1)
        bufs_b = (b0, b1)
        sems_a = (sa0, sa1)
        sems_b = (sb0, sb1)
        pltpu.sync_copy(src_i.at[w], idx_s)
        pltpu.sync_copy(dst_i.at[w], idx_d)

        def start(j, buf):
            pltpu.async_copy(table.at[idx_s.at[j]], bufs_a[buf], sems_a[buf])
            pltpu.async_copy(table.at[idx_d.at[j]], bufs_b[buf], sems_b[buf])

        def finish(j, buf):
            pltpu.make_async_copy(table.at[idx_s.at[j]], bufs_a[buf],
                                  sems_a[buf]).wait()
            pltpu.sync_copy(bufs_a[buf], out_a.at[w, j])
            pltpu.make_async_copy(table.at[idx_d.at[j]], bufs_b[buf],
                                  sems_b[buf]).wait()
            pltpu.sync_copy(bufs_b[buf], out_b.at[w, j])

        start(0, 0)

        def pair(p, carry):
            j0 = 2 * p
            start(j0 + 1, 1)
            finish(j0, 0)

            @pl.when(p < NCH // 2 - 1)
            def _():
                start(j0 + 2, 0)

            finish(j0 + 1, 1)
            return carry

        lax.fori_loop(0, NCH // 2, pair, 0)

    out = jax.ShapeDtypeStruct((NW, NCH, CH, feat_dim), dtype)
    return pl.kernel(
        body,
        out_type=[out, out],
        mesh=mesh,
        scratch_types=[
            pltpu.VMEM((NCH, CH), jnp.int32),
            pltpu.VMEM((NCH, CH), jnp.int32),
            pltpu.VMEM((CH, feat_dim), dtype),
            pltpu.VMEM((CH, feat_dim), dtype),
            pltpu.VMEM((CH, feat_dim), dtype),
            pltpu.VMEM((CH, feat_dim), dtype),
            pltpu.SemaphoreType.DMA,
            pltpu.SemaphoreType.DMA,
            pltpu.SemaphoreType.DMA,
            pltpu.SemaphoreType.DMA,
        ],
        compiler_params=pltpu.CompilerParams(use_tc_tiling_on_sc=tiled),
    )


@functools.cache
def _make_scatter():
    mesh = plsc.VectorSubcoreMesh(core_axis_name="c", subcore_axis_name="s", num_cores=NC, num_subcores=NS)

    def body(m3, dst_i, zeros, out, acc, idx_all, m0, m1, sm0, sm1):
        c = lax.axis_index("c")
        s = lax.axis_index("s")
        w = s * NC + c
        mbufs = (m0, m1)
        msems = (sm0, sm1)
        pltpu.sync_copy(zeros.at[pl.ds(s * STRIPE, STRIPE)],
                        acc.at[pl.ds(s * STRIPE, STRIPE)])
        pltpu.sync_copy(dst_i.at[w], idx_all)
        plsc.subcore_barrier()

        def start(j, buf):
            pltpu.async_copy(m3.at[w, j], mbufs[buf], msems[buf])

        def finish(j, buf):
            pltpu.make_async_copy(m3.at[w, j], mbufs[buf], msems[buf]).wait()
            pltpu.sync_copy(mbufs[buf], acc.at[idx_all.at[j]], add=True)

        start(0, 0)

        def pair(p, carry):
            j0 = 2 * p
            start(j0 + 1, 1)
            finish(j0, 0)

            @pl.when(p < NCH // 2 - 1)
            def _():
                start(j0 + 2, 0)

            finish(j0 + 1, 1)
            return carry

        lax.fori_loop(0, NCH // 2, pair, 0)
        plsc.subcore_barrier()
        pltpu.sync_copy(acc.at[pl.ds(s * STRIPE, STRIPE)],
                        out.at[c, pl.ds(s * STRIPE, STRIPE)])

    return pl.kernel(
        body,
        out_type=jax.ShapeDtypeStruct((NC, NA, M_DIM), _f32),
        mesh=mesh,
        scratch_types=[
            pltpu.VMEM_SHARED((NA, M_DIM), _f32),
            pltpu.VMEM((NCH, CH), jnp.int32),
            pltpu.VMEM((CH, M_DIM), _f32),
            pltpu.VMEM((CH, M_DIM), _f32),
            pltpu.SemaphoreType.DMA,
            pltpu.SemaphoreType.DMA,
        ],
        compiler_params=pltpu.CompilerParams(use_tc_tiling_on_sc=False),
    )


_BN = 400


def _lin0_body(x_ref, w_ref, b_ref, o_ref):
    o_ref[...] = jnp.maximum(
        jnp.dot(x_ref[...], w_ref[...], preferred_element_type=_f32)
        + b_ref[...], 0.0)


def _lin0(x, w, b):
    return pl.pallas_call(
        _lin0_body,
        grid=(N // _BN,),
        in_specs=[
            pl.BlockSpec((_BN, F), lambda i: (i, 0)),
            pl.BlockSpec((F, F), lambda i: (0, 0)),
            pl.BlockSpec((1, F), lambda i: (0, 0)),
        ],
        out_specs=pl.BlockSpec((_BN, F), lambda i: (i, 0)),
        out_shape=jax.ShapeDtypeStruct((N, F), _f32),
    )(x, w, b.reshape(1, F))


_CB = 4
_BE = _CB * CH


def _edge_body(use_relu, fd_ref, fs_ref, cs_ref, cd_ref, w1d_ref, w1s_ref,
               wl_ref, b1_ref, w2_ref, b2_ref, o_ref):
    fd = fd_ref[...].reshape(_BE, F)
    fs = fs_ref[...].reshape(_BE, F)
    cs = cs_ref[...].reshape(_BE, 16)
    cd = cd_ref[...].reshape(_BE, 16)
    if use_relu:
        cs = jnp.maximum(cs, 0.0)
        cd = jnp.maximum(cd, 0.0)
    d = cs - cd
    rel = jnp.sum(d * d, axis=1, keepdims=True)
    relb = rel.astype(jnp.bfloat16).astype(_f32)
    wlb = wl_ref[...].astype(jnp.bfloat16).astype(_f32)
    w1d = w1d_ref[...].astype(jnp.bfloat16)
    w1s = w1s_ref[...].astype(jnp.bfloat16)
    m1 = (jnp.dot(fd.astype(jnp.bfloat16), w1d, preferred_element_type=_f32)
          + jnp.dot(fs.astype(jnp.bfloat16), w1s, preferred_element_type=_f32)
          + relb * wlb + b1_ref[...])
    m1 = _silu(m1)
    m2 = jnp.dot(m1.astype(jnp.bfloat16),
                 w2_ref[...].astype(jnp.bfloat16),
                 preferred_element_type=_f32) + b2_ref[...]
    o_ref[...] = _silu(m2).reshape(1, _CB, CH, M_DIM)


def _edge(use_relu, fd4, fs4, cs4, cd4, ew1, eb1, ew2, eb2):
    body = functools.partial(_edge_body, use_relu)
    return pl.pallas_call(
        body,
        grid=(NW, NCH // _CB),
        in_specs=[
            pl.BlockSpec((1, _CB, CH, F), lambda w, j: (w, j, 0, 0)),
            pl.BlockSpec((1, _CB, CH, F), lambda w, j: (w, j, 0, 0)),
            pl.BlockSpec((1, _CB, CH, 16), lambda w, j: (w, j, 0, 0)),
            pl.BlockSpec((1, _CB, CH, 16), lambda w, j: (w, j, 0, 0)),
            pl.BlockSpec((F, EDGE_HID), lambda w, j: (0, 0)),
            pl.BlockSpec((F, EDGE_HID), lambda w, j: (0, 0)),
            pl.BlockSpec((1, EDGE_HID), lambda w, j: (0, 0)),
            pl.BlockSpec((1, EDGE_HID), lambda w, j: (0, 0)),
            pl.BlockSpec((EDGE_HID, M_DIM), lambda w, j: (0, 0)),
            pl.BlockSpec((1, M_DIM), lambda w, j: (0, 0)),
        ],
        out_specs=pl.BlockSpec((1, _CB, CH, M_DIM), lambda w, j: (w, j, 0, 0)),
        out_shape=jax.ShapeDtypeStruct((NW, NCH, CH, M_DIM), _f32),
    )(fd4, fs4, cs4, cd4, ew1[:F], ew1[F:2 * F], ew1[2 * F:2 * F + 1],
      eb1.reshape(1, EDGE_HID), ew2, eb2.reshape(1, M_DIM))


def _node_body(h_ref, mi_ref, w1f_ref, w1m_ref, b1_ref, w2_ref, b2_ref,
               o_ref):
    h = h_ref[...]
    m = mi_ref[0] + mi_ref[1]
    u = _silu(jnp.dot(h, w1f_ref[...], preferred_element_type=_f32)
              + jnp.dot(m, w1m_ref[...], preferred_element_type=_f32)
              + b1_ref[...])
    u = jnp.dot(u, w2_ref[...], preferred_element_type=_f32) + b2_ref[...]
    o_ref[...] = jnp.maximum(h + u, 0.0)


def _node(h, mi, nw1, nb1, nw2, nb2):
    return pl.pallas_call(
        _node_body,
        grid=(N // _BN,),
        in_specs=[
            pl.BlockSpec((_BN, F), lambda i: (i, 0)),
            pl.BlockSpec((NC, _BN, M_DIM), lambda i: (0, i, 0)),
            pl.BlockSpec((F, NODE_HID), lambda i: (0, 0)),
            pl.BlockSpec((M_DIM, NODE_HID), lambda i: (0, 0)),
            pl.BlockSpec((1, NODE_HID), lambda i: (0, 0)),
            pl.BlockSpec((NODE_HID, F), lambda i: (0, 0)),
            pl.BlockSpec((1, F), lambda i: (0, 0)),
        ],
        out_specs=pl.BlockSpec((_BN, F), lambda i: (i, 0)),
        out_shape=jax.ShapeDtypeStruct((N, F), _f32),
    )(h, mi, nw1[:F], nw1[F:], nb1.reshape(1, NODE_HID), nw2,
      nb2.reshape(1, F))


def _pool_body(h_ref, b_ref, o_ref):
    i = pl.program_id(0)

    @pl.when(i == 0)
    def _():
        o_ref[...] = jnp.zeros_like(o_ref)

    seg = b_ref[0]
    mask = (lax.broadcasted_iota(jnp.int32, (G, _BN), 0) == seg).astype(_f32)
    o_ref[...] += jnp.dot(mask, h_ref[...], preferred_element_type=_f32)


def _pool(h, batch3):
    return pl.pallas_call(
        _pool_body,
        grid=(N // _BN,),
        in_specs=[
            pl.BlockSpec((_BN, F), lambda i: (i, 0)),
            pl.BlockSpec((1, 1, _BN), lambda i: (i, 0, 0)),
        ],
        out_specs=pl.BlockSpec((G, F), lambda i: (0, 0)),
        out_shape=jax.ShapeDtypeStruct((G, F), _f32),
    )(h, batch3)


def _head_body(p0_ref, p1_ref, w1a_ref, w1b_ref, b1_ref, w2_ref, b2_ref,
               o_ref):
    z = (jnp.dot(p0_ref[...], w1a_ref[...], preferred_element_type=_f32)
         + jnp.dot(p1_ref[...], w1b_ref[...], preferred_element_type=_f32)
         + b1_ref[...])
    z = jnp.maximum(z, 0.0)
    o_ref[...] = jnp.dot(z, w2_ref[...], preferred_element_type=_f32) \
        + b2_ref[...]


def _head(p0, p1, w1, b1, w2, b2):
    return pl.pallas_call(
        _head_body,
        out_shape=jax.ShapeDtypeStruct((G, 1), _f32),
    )(p0, p1, w1[:F], w1[F:], b1.reshape(1, F), w2, b2.reshape(1, 1))


def _pad_idx(v, pad_val):
    v2 = v.reshape(NW, ET)
    v2 = jnp.pad(v2, ((0, 0), (0, ETP - ET)), constant_values=pad_val)
    return v2.reshape(NW, NCH, CH)


def kernel(x, coord, edge_index, batch, lin0_W, lin0_b, edge_W1, edge_b1,
           edge_W2, edge_b2, node_W1, node_b1, node_W2, node_b2, lin1_W,
           lin1_b, lin2_W, lin2_b):
    h0 = _lin0(x, lin0_W, lin0_b)
    coordp = jnp.pad(coord, ((0, 0), (0, 13)))
    zeros_acc = jnp.zeros((NA, M_DIM), _f32)
    batch3 = batch.astype(jnp.int32).reshape(N // _BN, 1, _BN)

    pooled = []
    for a in range(2):
        src = edge_index[a, 0].astype(jnp.int32)
        dst = edge_index[a, 1].astype(jnp.int32)
        src_g = _pad_idx(src, 0)
        dst_g = _pad_idx(dst, 0)
        dst_s = _pad_idx(dst, N)

        cs4, cd4 = _make_gather(16, _f32)(coordp, src_g, dst_g)

        h = h0
        for l in range(3):
            i = a * 3 + l
            fs4, fd4 = _make_gather(F, _f32, True)(h, src_g, dst_g)
            m4 = _edge(l > 0, fd4, fs4, cs4, cd4,
                       edge_W1[i], edge_b1[i], edge_W2[i], edge_b2[i])
            parts = _make_scatter()(m4, dst_s, zeros_acc)
            h = _node(h, parts, node_W1[i], node_b1[i],
                      node_W2[i], node_b2[i])
        pooled.append(_pool(h, batch3))

    return _head(pooled[0], pooled[1], lin1_W, lin1_b, lin2_W, lin2_b)
